# initial kernel scaffold (unmeasured)
import jax
import jax.numpy as jnp
from jax import lax
from jax.experimental import pallas as pl
from jax.experimental.pallas import tpu as pltpu

B, SQ, H, D = 4, 32, 8, 128
KV = 4096
Y = 4
SCALE = D ** -0.5


def _flash_body(q_ref, k_ref, v_ref, o_ref, m_ref, l_ref):
    q = q_ref[0, :, 0, :]
    k = k_ref[0, :, 0, :]
    v = v_ref[0, :, 0, :]
    s = lax.dot_general(
        q, k, (((1,), (1,)), ((), ())), preferred_element_type=jnp.float32
    ) * SCALE
    m = jnp.max(s, axis=1, keepdims=True)
    p = jnp.exp(s - m)
    l = jnp.sum(p, axis=1, keepdims=True)
    o = lax.dot_general(
        p, v, (((1,), (0,)), ((), ())), preferred_element_type=jnp.float32
    )
    o_ref[0, :, 0, :] = o
    m_ref[0, :, 0] = m[:, 0]
    l_ref[0, :, 0] = l[:, 0]


def _combine_body(o_ref, m_ref, l_ref, out_ref,
                  obuf, mbuf, lbuf, so, ro, sm, rm, sl, rl):
    my_x = lax.axis_index("x")
    my_y = lax.axis_index("y")
    my_z = lax.axis_index("z")
    left = (my_y - 1) % Y
    right = (my_y + 1) % Y

    barrier_sem = pltpu.get_barrier_semaphore()
    for nbr in (left, right):
        pl.semaphore_signal(
            barrier_sem, inc=1,
            device_id=(my_x, nbr, my_z),
            device_id_type=pl.DeviceIdType.MESH,
        )
    pl.semaphore_wait(barrier_sem, 2)

    obuf[0] = o_ref[...]
    mbuf[0] = m_ref[...]
    lbuf[0] = l_ref[...]

    for h in range(Y - 1):
        rdmas = []
        for buf, ss, rs in ((obuf, so, ro), (mbuf, sm, rm), (lbuf, sl, rl)):
            r = pltpu.make_async_remote_copy(
                src_ref=buf.at[h],
                dst_ref=buf.at[h + 1],
                send_sem=ss.at[h],
                recv_sem=rs.at[h],
                device_id=(my_x, right, my_z),
                device_id_type=pl.DeviceIdType.MESH,
            )
            r.start()
            rdmas.append(r)
        for r in rdmas:
            r.wait()

    m0, m1, m2, m3 = mbuf[0], mbuf[1], mbuf[2], mbuf[3]
    m = jnp.maximum(jnp.maximum(m0, m1), jnp.maximum(m2, m3))
    num = jnp.zeros((B, SQ, H, D), jnp.float32)
    den = jnp.zeros((B, SQ, H), jnp.float32)
    for j, mj in enumerate((m0, m1, m2, m3)):
        a = jnp.exp(mj - m)
        num = num + a[..., None] * obuf[j]
        den = den + a * lbuf[j]
    out_ref[...] = num / den[..., None]


def kernel(Q, K, V):
    o_u, m_, l_ = pl.pallas_call(
        _flash_body,
        grid=(B, H),
        in_specs=[
            pl.BlockSpec((1, SQ, 1, D), lambda b, h: (b, 0, h, 0)),
            pl.BlockSpec((1, KV, 1, D), lambda b, h: (b, 0, h, 0)),
            pl.BlockSpec((1, KV, 1, D), lambda b, h: (b, 0, h, 0)),
        ],
        out_specs=[
            pl.BlockSpec((1, SQ, 1, D), lambda b, h: (b, 0, h, 0)),
            pl.BlockSpec((1, SQ, 1), lambda b, h: (b, 0, h)),
            pl.BlockSpec((1, SQ, 1), lambda b, h: (b, 0, h)),
        ],
        out_shape=[
            jax.ShapeDtypeStruct((B, SQ, H, D), jnp.float32),
            jax.ShapeDtypeStruct((B, SQ, H), jnp.float32),
            jax.ShapeDtypeStruct((B, SQ, H), jnp.float32),
        ],
    )(Q, K, V)

    out = pl.pallas_call(
        _combine_body,
        in_specs=[
            pl.BlockSpec(memory_space=pltpu.VMEM),
            pl.BlockSpec(memory_space=pltpu.VMEM),
            pl.BlockSpec(memory_space=pltpu.VMEM),
        ],
        out_specs=pl.BlockSpec(memory_space=pltpu.VMEM),
        out_shape=jax.ShapeDtypeStruct((B, SQ, H, D), jnp.float32),
        scratch_shapes=[
            pltpu.VMEM((Y, B, SQ, H, D), jnp.float32),
            pltpu.VMEM((Y, B, SQ, H), jnp.float32),
            pltpu.VMEM((Y, B, SQ, H), jnp.float32),
            pltpu.SemaphoreType.DMA((Y - 1,)),
            pltpu.SemaphoreType.DMA((Y - 1,)),
            pltpu.SemaphoreType.DMA((Y - 1,)),
            pltpu.SemaphoreType.DMA((Y - 1,)),
            pltpu.SemaphoreType.DMA((Y - 1,)),
            pltpu.SemaphoreType.DMA((Y - 1,)),
        ],
        compiler_params=pltpu.CompilerParams(collective_id=0),
    )(o_u, m_, l_)
    return out


# baseline (device time: 131716 ns/iter reference)
import jax
import jax.numpy as jnp
from jax import lax
from jax.experimental import pallas as pl
from jax.experimental.pallas import tpu as pltpu

B, SQ, H, D = 4, 32, 8, 128
KV = 4096
Y = 4
SCALE = D ** -0.5


KVB = 1024
NKV = KV // KVB


def _flash_body(q_ref, k_ref, v_ref, o_ref, m_ref, l_ref, o_sc, m_sc, l_sc):
    kv_i = pl.program_id(1)

    @pl.when(kv_i == 0)
    def _():
        m_sc[...] = jnp.full((H, SQ, 1), -1e30, jnp.float32)
        l_sc[...] = jnp.zeros((H, SQ, 1), jnp.float32)
        o_sc[...] = jnp.zeros((H, SQ, D), jnp.float32)

    for h in range(H):
        q = q_ref[0, :, h, :]
        k = k_ref[0, :, h, :]
        v = v_ref[0, :, h, :]
        s = lax.dot_general(
            q, k, (((1,), (1,)), ((), ())), preferred_element_type=jnp.float32
        ) * SCALE
        m_old = m_sc[h]
        m_new = jnp.maximum(m_old, jnp.max(s, axis=1, keepdims=True))
        alpha = jnp.exp(m_old - m_new)
        p = jnp.exp(s - m_new)
        l_sc[h] = alpha * l_sc[h] + jnp.sum(p, axis=1, keepdims=True)
        o_sc[h] = alpha * o_sc[h] + lax.dot_general(
            p, v, (((1,), (0,)), ((), ())), preferred_element_type=jnp.float32
        )
        m_sc[h] = m_new

    @pl.when(kv_i == NKV - 1)
    def _():
        o_ref[0] = jnp.stack([o_sc[h] for h in range(H)], axis=1)
        m_ref[0] = jnp.stack([m_sc[h][:, 0] for h in range(H)], axis=1)
        l_ref[0] = jnp.stack([l_sc[h][:, 0] for h in range(H)], axis=1)


def _combine_body(o_ref, m_ref, l_ref, out_ref,
                  obuf, mbuf, lbuf, so, ro, sm, rm, sl, rl):
    my_x = lax.axis_index("x")
    my_y = lax.axis_index("y")
    my_z = lax.axis_index("z")
    left = (my_y - 1) % Y
    right = (my_y + 1) % Y

    barrier_sem = pltpu.get_barrier_semaphore()
    for nbr in (left, right):
        pl.semaphore_signal(
            barrier_sem, inc=1,
            device_id=(my_x, nbr, my_z),
            device_id_type=pl.DeviceIdType.MESH,
        )
    pl.semaphore_wait(barrier_sem, 2)

    obuf[0] = o_ref[...]
    mbuf[0] = m_ref[...]
    lbuf[0] = l_ref[...]

    for h in range(Y - 1):
        rdmas = []
        for buf, ss, rs in ((obuf, so, ro), (mbuf, sm, rm), (lbuf, sl, rl)):
            r = pltpu.make_async_remote_copy(
                src_ref=buf.at[h],
                dst_ref=buf.at[h + 1],
                send_sem=ss.at[h],
                recv_sem=rs.at[h],
                device_id=(my_x, right, my_z),
                device_id_type=pl.DeviceIdType.MESH,
            )
            r.start()
            rdmas.append(r)
        for r in rdmas:
            r.wait()

    m0, m1, m2, m3 = mbuf[0], mbuf[1], mbuf[2], mbuf[3]
    m = jnp.maximum(jnp.maximum(m0, m1), jnp.maximum(m2, m3))
    num = jnp.zeros((B, SQ, H, D), jnp.float32)
    den = jnp.zeros((B, SQ, H), jnp.float32)
    for j, mj in enumerate((m0, m1, m2, m3)):
        a = jnp.exp(mj - m)
        num = num + a[..., None] * obuf[j]
        den = den + a * lbuf[j]
    out_ref[...] = num / den[..., None]


def kernel(Q, K, V):
    o_u, m_, l_ = pl.pallas_call(
        _flash_body,
        grid=(B, NKV),
        in_specs=[
            pl.BlockSpec((1, SQ, H, D), lambda b, i: (b, 0, 0, 0)),
            pl.BlockSpec((1, KVB, H, D), lambda b, i: (b, i, 0, 0)),
            pl.BlockSpec((1, KVB, H, D), lambda b, i: (b, i, 0, 0)),
        ],
        out_specs=[
            pl.BlockSpec((1, SQ, H, D), lambda b, i: (b, 0, 0, 0)),
            pl.BlockSpec((1, SQ, H), lambda b, i: (b, 0, 0)),
            pl.BlockSpec((1, SQ, H), lambda b, i: (b, 0, 0)),
        ],
        out_shape=[
            jax.ShapeDtypeStruct((B, SQ, H, D), jnp.float32),
            jax.ShapeDtypeStruct((B, SQ, H), jnp.float32),
            jax.ShapeDtypeStruct((B, SQ, H), jnp.float32),
        ],
        scratch_shapes=[
            pltpu.VMEM((H, SQ, D), jnp.float32),
            pltpu.VMEM((H, SQ, 1), jnp.float32),
            pltpu.VMEM((H, SQ, 1), jnp.float32),
        ],
    )(Q, K, V)

    out = pl.pallas_call(
        _combine_body,
        in_specs=[
            pl.BlockSpec(memory_space=pltpu.VMEM),
            pl.BlockSpec(memory_space=pltpu.VMEM),
            pl.BlockSpec(memory_space=pltpu.VMEM),
        ],
        out_specs=pl.BlockSpec(memory_space=pltpu.VMEM),
        out_shape=jax.ShapeDtypeStruct((B, SQ, H, D), jnp.float32),
        scratch_shapes=[
            pltpu.VMEM((Y, B, SQ, H, D), jnp.float32),
            pltpu.VMEM((Y, B, SQ, H), jnp.float32),
            pltpu.VMEM((Y, B, SQ, H), jnp.float32),
            pltpu.SemaphoreType.DMA((Y - 1,)),
            pltpu.SemaphoreType.DMA((Y - 1,)),
            pltpu.SemaphoreType.DMA((Y - 1,)),
            pltpu.SemaphoreType.DMA((Y - 1,)),
            pltpu.SemaphoreType.DMA((Y - 1,)),
            pltpu.SemaphoreType.DMA((Y - 1,)),
        ],
        compiler_params=pltpu.CompilerParams(collective_id=0),
    )(o_u, m_, l_)
    return out
